# baseline (speedup 1.0000x reference)
import functools

import jax
import jax.numpy as jnp
from jax import lax
from jax.experimental import pallas as pl
from jax.experimental.pallas import tpu as pltpu
from jax.experimental.pallas import tpu_sc as plsc

N = 10000
NP = 10240
E = 320000
D = 128
NQ = 8

NC = 2
NS = 16
NW = NC * NS
K = 128
NCH = 80
EPT = NCH * K
EP = NW * EPT
NBUF = 2
ROWS_PT = NP // NS

_MESH = dict(core_axis_name="c", subcore_axis_name="s",
             num_cores=NC, num_subcores=NS)


def _deg_body(dste_hbm, ones_hbm, zdeg_hbm, out_hbm, acc, dstb, ones_v):
    c = lax.axis_index("c")
    s = lax.axis_index("s")

    @pl.when(s == 0)
    def _():
        pltpu.sync_copy(zdeg_hbm, acc)

    wid = c * NS + s
    pltpu.sync_copy(dste_hbm.at[wid], dstb)
    pltpu.sync_copy(ones_hbm, ones_v)
    plsc.subcore_barrier()

    def body(j, carry):
        pltpu.sync_copy(ones_v, acc.at[dstb.at[j]], add=True)
        return carry

    lax.fori_loop(0, NCH, body, 0)
    plsc.subcore_barrier()

    @pl.when(s == 0)
    def _():
        pltpu.sync_copy(acc, out_hbm.at[pl.ds(c * NP, NP)])


def _spmm_body(y_hbm, srce_hbm, dste_hbm, z2d_hbm, out_hbm,
               acc, src_bufs, dst_bufs, isems, rows_bufs, gsems):
    c = lax.axis_index("c")
    s = lax.axis_index("s")
    wid = c * NS + s

    pltpu.sync_copy(z2d_hbm, acc.at[pl.ds(s * ROWS_PT, ROWS_PT)])
    plsc.subcore_barrier()

    ebase = wid * EPT
    bufs = tuple(zip(rows_bufs, gsems, src_bufs, dst_bufs, isems))

    def fire_idx(ch, srcv, dstv, isem):
        pltpu.async_copy(srce_hbm.at[pl.ds(ebase + ch * K, K)], srcv, isem)
        pltpu.async_copy(dste_hbm.at[pl.ds(ebase + ch * K, K)], dstv, isem)

    def wait_idx(srcv, dstv, isem):
        pltpu.make_async_copy(srce_hbm.at[pl.ds(ebase, K)], srcv, isem).wait()
        pltpu.make_async_copy(dste_hbm.at[pl.ds(ebase, K)], dstv, isem).wait()

    for b, (rows, gsem, srcv, dstv, isem) in enumerate(bufs):
        fire_idx(b, srcv, dstv, isem)
    wait_idx(*bufs[0][2:])
    pltpu.async_copy(y_hbm.at[bufs[0][2]], bufs[0][0], bufs[0][1])

    def body(i, carry):
        for b, (rows, gsem, srcv, dstv, isem) in enumerate(bufs):
            ch = i * NBUF + b
            bn = (b + 1) % NBUF
            rows_n, gsem_n, srcv_n, dstv_n, isem_n = bufs[bn]
            pltpu.make_async_copy(y_hbm.at[srcv], rows, gsem).wait()

            @pl.when(ch + 1 < NCH)
            def _():
                wait_idx(srcv_n, dstv_n, isem_n)
                pltpu.async_copy(y_hbm.at[srcv_n], rows_n, gsem_n)

            pltpu.sync_copy(rows, acc.at[dstv], add=True)

            @pl.when(ch + NBUF < NCH)
            def _():
                fire_idx(ch + NBUF, srcv, dstv, isem)
        return carry

    lax.fori_loop(0, NCH // NBUF, body, 0)
    plsc.subcore_barrier()

    pltpu.sync_copy(acc.at[pl.ds(s * ROWS_PT, ROWS_PT)],
                    out_hbm.at[c, pl.ds(s * ROWS_PT, ROWS_PT)])


@functools.cache
def _get_deg_kernel():
    return pl.kernel(
        _deg_body,
        out_type=jax.ShapeDtypeStruct((NC * NP,), jnp.float32),
        mesh=plsc.VectorSubcoreMesh(**_MESH),
        scratch_types=[
            pltpu.VMEM_SHARED((NP,), jnp.float32),
            pltpu.VMEM((NCH, K), jnp.int32),
            pltpu.VMEM((K,), jnp.float32),
        ],
    )


@functools.cache
def _get_spmm_kernel():
    return pl.kernel(
        _spmm_body,
        out_type=jax.ShapeDtypeStruct((NC, NP, D), jnp.float32),
        mesh=plsc.VectorSubcoreMesh(**_MESH),
        scratch_types=[
            pltpu.VMEM_SHARED((NP, D), jnp.float32),
            [pltpu.VMEM((K,), jnp.int32)] * NBUF,
            [pltpu.VMEM((K,), jnp.int32)] * NBUF,
            [pltpu.SemaphoreType.DMA] * NBUF,
            [pltpu.VMEM((K, D), jnp.float32)] * NBUF,
            [pltpu.SemaphoreType.DMA] * NBUF,
        ],
    )


R = 1024


def _dinv(d0, d1):
    return lax.rsqrt(d0 + d1 + 1.0)


def _prep1_body(d0_ref, d1_ref, x_ref, w_ref, o_ref):
    dinv = _dinv(d0_ref[...], d1_ref[...])
    o_ref[...] = dinv * jnp.dot(x_ref[...], w_ref[...],
                                preferred_element_type=jnp.float32)


def _mid_body(d0_ref, d1_ref, a0_ref, a1_ref, y1_ref, b1_ref, w2_ref,
              q_ref, wm_ref, bm_ref, o_ref, ques_ref):
    dinv = _dinv(d0_ref[...], d1_ref[...])
    h1 = dinv * (a0_ref[...] + a1_ref[...] + y1_ref[...]) + b1_ref[...]
    h1 = jnp.maximum(h1, 0.0)
    o_ref[...] = dinv * jnp.dot(h1, w2_ref[...],
                                preferred_element_type=jnp.float32)

    @pl.when(pl.program_id(0) == 0)
    def _():
        ques_ref[...] = jnp.maximum(
            jnp.dot(q_ref[...], wm_ref[...],
                    preferred_element_type=jnp.float32) + bm_ref[...], 0.0)


def _final_body(d0_ref, d1_ref, a0_ref, a1_ref, y2_ref, b2_ref, o_ref):
    dinv = _dinv(d0_ref[...], d1_ref[...])
    o_ref[...] = dinv * (a0_ref[...] + a1_ref[...] + y2_ref[...]) + b2_ref[...]


def _col_spec():
    return pl.BlockSpec((R, 1), lambda i: (i, 0))


def _row_spec():
    return pl.BlockSpec((R, D), lambda i: (i, 0))


def _full_spec(shape):
    return pl.BlockSpec(shape, lambda i: (0,) * len(shape))


_prep1 = pl.pallas_call(
    _prep1_body,
    grid=(NP // R,),
    in_specs=[_col_spec(), _col_spec(), _row_spec(), _full_spec((D, D))],
    out_specs=_row_spec(),
    out_shape=jax.ShapeDtypeStruct((NP, D), jnp.float32),
)

_mid = pl.pallas_call(
    _mid_body,
    grid=(NP // R,),
    in_specs=[_col_spec(), _col_spec(), _row_spec(), _row_spec(), _row_spec(),
              _full_spec((1, D)), _full_spec((D, D)),
              _full_spec((NQ, D)), _full_spec((D, D)), _full_spec((1, D))],
    out_specs=[_row_spec(), _full_spec((NQ, D))],
    out_shape=[jax.ShapeDtypeStruct((NP, D), jnp.float32),
               jax.ShapeDtypeStruct((NQ, D), jnp.float32)],
)

_final = pl.pallas_call(
    _final_body,
    grid=(NP // R,),
    in_specs=[_col_spec(), _col_spec(), _row_spec(), _row_spec(), _row_spec(),
              _full_spec((1, D))],
    out_specs=_row_spec(),
    out_shape=jax.ShapeDtypeStruct((NP, D), jnp.float32),
)


def kernel(x, edge_index, W1, b1, W2, b2, q_emb, Wm, bm):
    pad = N + (jnp.arange(EP - E, dtype=jnp.int32) % (NP - N))
    srcf = jnp.concatenate([edge_index[0], pad])
    dstf = jnp.concatenate([edge_index[1], pad])
    dstp = dstf.reshape(NW, NCH, K)
    xp = jnp.pad(x, ((0, NP - N), (0, 0)))

    zdeg = jnp.zeros((NP,), jnp.float32)
    z2d = jnp.zeros((ROWS_PT, D), jnp.float32)
    ones_k = jnp.ones((K,), jnp.float32)

    degp = _get_deg_kernel()(dstp, ones_k, zdeg)
    d0 = degp[:NP].reshape(NP, 1)
    d1 = degp[NP:].reshape(NP, 1)

    y1 = _prep1(d0, d1, xp, W1)
    acc1 = _get_spmm_kernel()(y1, srcf, dstf, z2d)
    y2, ques = _mid(d0, d1, acc1[0], acc1[1], y1, b1.reshape(1, D), W2,
                    q_emb, Wm, bm.reshape(1, D))
    acc2 = _get_spmm_kernel()(y2, srcf, dstf, z2d)
    h2 = _final(d0, d1, acc2[0], acc2[1], y2, b2.reshape(1, D))
    return (ques, h2[:N])


# speedup vs baseline: 1.0015x; 1.0015x over previous
import functools

import jax
import jax.numpy as jnp
from jax import lax
from jax.experimental import pallas as pl
from jax.experimental.pallas import tpu as pltpu
from jax.experimental.pallas import tpu_sc as plsc

N = 10000
NP = 10240
E = 320000
D = 128
NQ = 8

NC = 2
NS = 16
NW = NC * NS
K = 128
NCH = 80
EPT = NCH * K
EP = NW * EPT
NIDX = 4
ROWS_PT = NP // NS

_MESH = dict(core_axis_name="c", subcore_axis_name="s",
             num_cores=NC, num_subcores=NS)


def _deg_body(dste_hbm, ones_hbm, zdeg_hbm, out_hbm, acc, dstb, ones_v):
    c = lax.axis_index("c")
    s = lax.axis_index("s")

    @pl.when(s == 0)
    def _():
        pltpu.sync_copy(zdeg_hbm, acc)

    wid = c * NS + s
    pltpu.sync_copy(dste_hbm.at[wid], dstb)
    pltpu.sync_copy(ones_hbm, ones_v)
    plsc.subcore_barrier()

    def body(j, carry):
        pltpu.sync_copy(ones_v, acc.at[dstb.at[j]], add=True)
        return carry

    lax.fori_loop(0, NCH, body, 0)
    plsc.subcore_barrier()

    @pl.when(s == 0)
    def _():
        pltpu.sync_copy(acc, out_hbm.at[pl.ds(c * NP, NP)])


def _spmm_body(y_hbm, srce_hbm, dste_hbm, z2d_hbm, out_hbm,
               acc, src_bufs, dst_bufs, isems, rows_bufs, gsems, ssems):
    c = lax.axis_index("c")
    s = lax.axis_index("s")
    wid = c * NS + s

    pltpu.sync_copy(z2d_hbm, acc.at[pl.ds(s * ROWS_PT, ROWS_PT)])
    plsc.subcore_barrier()

    ebase = wid * EPT
    idx_sets = tuple(zip(src_bufs, dst_bufs, isems))

    def fire_idx(ch, srcv, dstv, isem):
        pltpu.async_copy(srce_hbm.at[pl.ds(ebase + ch * K, K)], srcv, isem)
        pltpu.async_copy(dste_hbm.at[pl.ds(ebase + ch * K, K)], dstv, isem)

    def wait_idx(srcv, dstv, isem):
        pltpu.make_async_copy(srce_hbm.at[pl.ds(ebase, K)], srcv, isem).wait()
        pltpu.make_async_copy(dste_hbm.at[pl.ds(ebase, K)], dstv, isem).wait()

    fire_idx(0, *idx_sets[0])
    fire_idx(1, *idx_sets[1])
    wait_idx(*idx_sets[0])
    pltpu.async_copy(y_hbm.at[idx_sets[0][0]], rows_bufs[0], gsems[0])

    def body(i, carry):
        for j in range(NIDX):
            ch = i * NIDX + j
            b, bn = j % 2, (j + 1) % 2
            srcv, dstv, isem = idx_sets[j]
            pltpu.make_async_copy(y_hbm.at[srcv], rows_bufs[b], gsems[b]).wait()
            pltpu.async_copy(rows_bufs[b], acc.at[dstv], ssems[b], add=True)

            @pl.when(ch > 0)
            def _():
                pltpu.make_async_copy(
                    rows_bufs[bn], acc.at[dstv], ssems[bn]).wait()

            @pl.when(ch + 1 < NCH)
            def _():
                wait_idx(*idx_sets[(j + 1) % NIDX])
                pltpu.async_copy(y_hbm.at[idx_sets[(j + 1) % NIDX][0]],
                                 rows_bufs[bn], gsems[bn])

            @pl.when(ch + 2 < NCH)
            def _():
                fire_idx(ch + 2, *idx_sets[(j + 2) % NIDX])
        return carry

    lax.fori_loop(0, NCH // NIDX, body, 0)
    pltpu.make_async_copy(rows_bufs[1], acc.at[idx_sets[0][1]], ssems[1]).wait()
    plsc.subcore_barrier()

    pltpu.sync_copy(acc.at[pl.ds(s * ROWS_PT, ROWS_PT)],
                    out_hbm.at[c, pl.ds(s * ROWS_PT, ROWS_PT)])


@functools.cache
def _get_deg_kernel():
    return pl.kernel(
        _deg_body,
        out_type=jax.ShapeDtypeStruct((NC * NP,), jnp.float32),
        mesh=plsc.VectorSubcoreMesh(**_MESH),
        scratch_types=[
            pltpu.VMEM_SHARED((NP,), jnp.float32),
            pltpu.VMEM((NCH, K), jnp.int32),
            pltpu.VMEM((K,), jnp.float32),
        ],
    )


@functools.cache
def _get_spmm_kernel():
    return pl.kernel(
        _spmm_body,
        out_type=jax.ShapeDtypeStruct((NC, NP, D), jnp.float32),
        mesh=plsc.VectorSubcoreMesh(**_MESH),
        scratch_types=[
            pltpu.VMEM_SHARED((NP, D), jnp.float32),
            [pltpu.VMEM((K,), jnp.int32)] * NIDX,
            [pltpu.VMEM((K,), jnp.int32)] * NIDX,
            [pltpu.SemaphoreType.DMA] * NIDX,
            [pltpu.VMEM((K, D), jnp.float32)] * 2,
            [pltpu.SemaphoreType.DMA] * 2,
            [pltpu.SemaphoreType.DMA] * 2,
        ],
    )


R = 1024


def _dinv(d0, d1):
    return lax.rsqrt(d0 + d1 + 1.0)


def _prep1_body(d0_ref, d1_ref, x_ref, w_ref, o_ref):
    dinv = _dinv(d0_ref[...], d1_ref[...])
    o_ref[...] = dinv * jnp.dot(x_ref[...], w_ref[...],
                                preferred_element_type=jnp.float32)


def _mid_body(d0_ref, d1_ref, a0_ref, a1_ref, y1_ref, b1_ref, w2_ref,
              q_ref, wm_ref, bm_ref, o_ref, ques_ref):
    dinv = _dinv(d0_ref[...], d1_ref[...])
    h1 = dinv * (a0_ref[...] + a1_ref[...] + y1_ref[...]) + b1_ref[...]
    h1 = jnp.maximum(h1, 0.0)
    o_ref[...] = dinv * jnp.dot(h1, w2_ref[...],
                                preferred_element_type=jnp.float32)

    @pl.when(pl.program_id(0) == 0)
    def _():
        ques_ref[...] = jnp.maximum(
            jnp.dot(q_ref[...], wm_ref[...],
                    preferred_element_type=jnp.float32) + bm_ref[...], 0.0)


def _final_body(d0_ref, d1_ref, a0_ref, a1_ref, y2_ref, b2_ref, o_ref):
    dinv = _dinv(d0_ref[...], d1_ref[...])
    o_ref[...] = dinv * (a0_ref[...] + a1_ref[...] + y2_ref[...]) + b2_ref[...]


def _col_spec():
    return pl.BlockSpec((R, 1), lambda i: (i, 0))


def _row_spec():
    return pl.BlockSpec((R, D), lambda i: (i, 0))


def _full_spec(shape):
    return pl.BlockSpec(shape, lambda i: (0,) * len(shape))


_prep1 = pl.pallas_call(
    _prep1_body,
    grid=(NP // R,),
    in_specs=[_col_spec(), _col_spec(), _row_spec(), _full_spec((D, D))],
    out_specs=_row_spec(),
    out_shape=jax.ShapeDtypeStruct((NP, D), jnp.float32),
)

_mid = pl.pallas_call(
    _mid_body,
    grid=(NP // R,),
    in_specs=[_col_spec(), _col_spec(), _row_spec(), _row_spec(), _row_spec(),
              _full_spec((1, D)), _full_spec((D, D)),
              _full_spec((NQ, D)), _full_spec((D, D)), _full_spec((1, D))],
    out_specs=[_row_spec(), _full_spec((NQ, D))],
    out_shape=[jax.ShapeDtypeStruct((NP, D), jnp.float32),
               jax.ShapeDtypeStruct((NQ, D), jnp.float32)],
)

_final = pl.pallas_call(
    _final_body,
    grid=(NP // R,),
    in_specs=[_col_spec(), _col_spec(), _row_spec(), _row_spec(), _row_spec(),
              _full_spec((1, D))],
    out_specs=_row_spec(),
    out_shape=jax.ShapeDtypeStruct((NP, D), jnp.float32),
)


def kernel(x, edge_index, W1, b1, W2, b2, q_emb, Wm, bm):
    pad = N + (jnp.arange(EP - E, dtype=jnp.int32) % (NP - N))
    srcf = jnp.concatenate([edge_index[0], pad])
    dstf = jnp.concatenate([edge_index[1], pad])
    dstp = dstf.reshape(NW, NCH, K)
    xp = jnp.pad(x, ((0, NP - N), (0, 0)))

    zdeg = jnp.zeros((NP,), jnp.float32)
    z2d = jnp.zeros((ROWS_PT, D), jnp.float32)
    ones_k = jnp.ones((K,), jnp.float32)

    degp = _get_deg_kernel()(dstp, ones_k, zdeg)
    d0 = degp[:NP].reshape(NP, 1)
    d1 = degp[NP:].reshape(NP, 1)

    y1 = _prep1(d0, d1, xp, W1)
    acc1 = _get_spmm_kernel()(y1, srcf, dstf, z2d)
    y2, ques = _mid(d0, d1, acc1[0], acc1[1], y1, b1.reshape(1, D), W2,
                    q_emb, Wm, bm.reshape(1, D))
    acc2 = _get_spmm_kernel()(y2, srcf, dstf, z2d)
    h2 = _final(d0, d1, acc2[0], acc2[1], y2, b2.reshape(1, D))
    return (ques, h2[:N])


# speedup vs baseline: 1.0618x; 1.0602x over previous
import functools

import jax
import jax.numpy as jnp
from jax import lax
from jax.experimental import pallas as pl
from jax.experimental.pallas import tpu as pltpu
from jax.experimental.pallas import tpu_sc as plsc

N = 10000
NP = 10240
E = 320000
D = 128
NQ = 8

NC = 2
NS = 16
NW = NC * NS
K = 96
NCH = 108
EPT = NCH * K
EP = NW * EPT
NIDX = 6
NROW = 3
UNROLL = 6
ROWS_PT = NP // NS

_MESH = dict(core_axis_name="c", subcore_axis_name="s",
             num_cores=NC, num_subcores=NS)


def _deg_body(dste_hbm, ones_hbm, zdeg_hbm, out_hbm, acc, dstb, ones_v):
    c = lax.axis_index("c")
    s = lax.axis_index("s")

    @pl.when(s == 0)
    def _():
        pltpu.sync_copy(zdeg_hbm, acc)

    wid = c * NS + s
    pltpu.sync_copy(dste_hbm.at[wid], dstb)
    pltpu.sync_copy(ones_hbm, ones_v)
    plsc.subcore_barrier()

    def body(j, carry):
        pltpu.sync_copy(ones_v, acc.at[dstb.at[j]], add=True)
        return carry

    lax.fori_loop(0, NCH, body, 0)
    plsc.subcore_barrier()

    @pl.when(s == 0)
    def _():
        pltpu.sync_copy(acc, out_hbm.at[pl.ds(c * NP, NP)])


def _spmm_body(y_hbm, srce_hbm, dste_hbm, z2d_hbm, out_hbm,
               acc, src_bufs, dst_bufs, isems, rows_bufs, gsems, ssems):
    c = lax.axis_index("c")
    s = lax.axis_index("s")
    wid = c * NS + s

    pltpu.sync_copy(z2d_hbm, acc.at[pl.ds(s * ROWS_PT, ROWS_PT)])
    plsc.subcore_barrier()

    ebase = wid * EPT
    idx_sets = tuple(zip(src_bufs, dst_bufs, isems))

    def fire_idx(ch, srcv, dstv, isem):
        pltpu.async_copy(srce_hbm.at[pl.ds(ebase + ch * K, K)], srcv, isem)
        pltpu.async_copy(dste_hbm.at[pl.ds(ebase + ch * K, K)], dstv, isem)

    def wait_idx(srcv, dstv, isem):
        pltpu.make_async_copy(srce_hbm.at[pl.ds(ebase, K)], srcv, isem).wait()
        pltpu.make_async_copy(dste_hbm.at[pl.ds(ebase, K)], dstv, isem).wait()

    def wait_scatter(b):
        pltpu.make_async_copy(rows_bufs[b], acc.at[idx_sets[0][1]],
                              ssems[b]).wait()

    fire_idx(0, *idx_sets[0])
    fire_idx(1, *idx_sets[1])
    wait_idx(*idx_sets[0])
    pltpu.async_copy(y_hbm.at[idx_sets[0][0]], rows_bufs[0], gsems[0])

    def body(i, carry):
        for j in range(UNROLL):
            ch = i * UNROLL + j
            b = j % NROW
            srcv, dstv, isem = idx_sets[j % NIDX]

            @pl.when(ch >= 2)
            def _():
                wait_scatter((j + 1) % NROW)

            @pl.when(ch + 1 < NCH)
            def _():
                wait_idx(*idx_sets[(j + 1) % NIDX])
                pltpu.async_copy(y_hbm.at[idx_sets[(j + 1) % NIDX][0]],
                                 rows_bufs[(j + 1) % NROW],
                                 gsems[(j + 1) % NROW])

            pltpu.make_async_copy(y_hbm.at[srcv], rows_bufs[b], gsems[b]).wait()
            pltpu.async_copy(rows_bufs[b], acc.at[dstv], ssems[b], add=True)

            @pl.when(ch + 2 < NCH)
            def _():
                fire_idx(ch + 2, *idx_sets[(j + 2) % NIDX])
        return carry

    lax.fori_loop(0, NCH // UNROLL, body, 0)
    wait_scatter((NCH - 2) % NROW)
    wait_scatter((NCH - 1) % NROW)
    plsc.subcore_barrier()

    pltpu.sync_copy(acc.at[pl.ds(s * ROWS_PT, ROWS_PT)],
                    out_hbm.at[c, pl.ds(s * ROWS_PT, ROWS_PT)])


@functools.cache
def _get_deg_kernel():
    return pl.kernel(
        _deg_body,
        out_type=jax.ShapeDtypeStruct((NC * NP,), jnp.float32),
        mesh=plsc.VectorSubcoreMesh(**_MESH),
        scratch_types=[
            pltpu.VMEM_SHARED((NP,), jnp.float32),
            pltpu.VMEM((NCH, K), jnp.int32),
            pltpu.VMEM((K,), jnp.float32),
        ],
    )


@functools.cache
def _get_spmm_kernel():
    return pl.kernel(
        _spmm_body,
        out_type=jax.ShapeDtypeStruct((NC, NP, D), jnp.float32),
        mesh=plsc.VectorSubcoreMesh(**_MESH),
        scratch_types=[
            pltpu.VMEM_SHARED((NP, D), jnp.float32),
            [pltpu.VMEM((K,), jnp.int32)] * NIDX,
            [pltpu.VMEM((K,), jnp.int32)] * NIDX,
            [pltpu.SemaphoreType.DMA] * NIDX,
            [pltpu.VMEM((K, D), jnp.float32)] * NROW,
            [pltpu.SemaphoreType.DMA] * NROW,
            [pltpu.SemaphoreType.DMA] * NROW,
        ],
    )


R = 1024


def _dinv(d0, d1):
    return lax.rsqrt(d0 + d1 + 1.0)


def _prep1_body(d0_ref, d1_ref, x_ref, w_ref, o_ref):
    dinv = _dinv(d0_ref[...], d1_ref[...])
    o_ref[...] = dinv * jnp.dot(x_ref[...], w_ref[...],
                                preferred_element_type=jnp.float32)


def _mid_body(d0_ref, d1_ref, a0_ref, a1_ref, y1_ref, b1_ref, w2_ref,
              q_ref, wm_ref, bm_ref, o_ref, ques_ref):
    dinv = _dinv(d0_ref[...], d1_ref[...])
    h1 = dinv * (a0_ref[...] + a1_ref[...] + y1_ref[...]) + b1_ref[...]
    h1 = jnp.maximum(h1, 0.0)
    o_ref[...] = dinv * jnp.dot(h1, w2_ref[...],
                                preferred_element_type=jnp.float32)

    @pl.when(pl.program_id(0) == 0)
    def _():
        ques_ref[...] = jnp.maximum(
            jnp.dot(q_ref[...], wm_ref[...],
                    preferred_element_type=jnp.float32) + bm_ref[...], 0.0)


def _final_body(d0_ref, d1_ref, a0_ref, a1_ref, y2_ref, b2_ref, o_ref):
    dinv = _dinv(d0_ref[...], d1_ref[...])
    o_ref[...] = dinv * (a0_ref[...] + a1_ref[...] + y2_ref[...]) + b2_ref[...]


def _col_spec():
    return pl.BlockSpec((R, 1), lambda i: (i, 0))


def _row_spec():
    return pl.BlockSpec((R, D), lambda i: (i, 0))


def _full_spec(shape):
    return pl.BlockSpec(shape, lambda i: (0,) * len(shape))


_prep1 = pl.pallas_call(
    _prep1_body,
    grid=(NP // R,),
    in_specs=[_col_spec(), _col_spec(), _row_spec(), _full_spec((D, D))],
    out_specs=_row_spec(),
    out_shape=jax.ShapeDtypeStruct((NP, D), jnp.float32),
)

_mid = pl.pallas_call(
    _mid_body,
    grid=(NP // R,),
    in_specs=[_col_spec(), _col_spec(), _row_spec(), _row_spec(), _row_spec(),
              _full_spec((1, D)), _full_spec((D, D)),
              _full_spec((NQ, D)), _full_spec((D, D)), _full_spec((1, D))],
    out_specs=[_row_spec(), _full_spec((NQ, D))],
    out_shape=[jax.ShapeDtypeStruct((NP, D), jnp.float32),
               jax.ShapeDtypeStruct((NQ, D), jnp.float32)],
)

_final = pl.pallas_call(
    _final_body,
    grid=(NP // R,),
    in_specs=[_col_spec(), _col_spec(), _row_spec(), _row_spec(), _row_spec(),
              _full_spec((1, D))],
    out_specs=_row_spec(),
    out_shape=jax.ShapeDtypeStruct((NP, D), jnp.float32),
)


def kernel(x, edge_index, W1, b1, W2, b2, q_emb, Wm, bm):
    pad = N + (jnp.arange(EP - E, dtype=jnp.int32) % (NP - N))
    srcf = jnp.concatenate([edge_index[0], pad])
    dstf = jnp.concatenate([edge_index[1], pad])
    dstp = dstf.reshape(NW, NCH, K)
    xp = jnp.pad(x, ((0, NP - N), (0, 0)))

    zdeg = jnp.zeros((NP,), jnp.float32)
    z2d = jnp.zeros((ROWS_PT, D), jnp.float32)
    ones_k = jnp.ones((K,), jnp.float32)

    degp = _get_deg_kernel()(dstp, ones_k, zdeg)
    d0 = degp[:NP].reshape(NP, 1)
    d1 = degp[NP:].reshape(NP, 1)

    y1 = _prep1(d0, d1, xp, W1)
    acc1 = _get_spmm_kernel()(y1, srcf, dstf, z2d)
    y2, ques = _mid(d0, d1, acc1[0], acc1[1], y1, b1.reshape(1, D), W2,
                    q_emb, Wm, bm.reshape(1, D))
    acc2 = _get_spmm_kernel()(y2, srcf, dstf, z2d)
    h2 = _final(d0, d1, acc2[0], acc2[1], y2, b2.reshape(1, D))
    return (ques, h2[:N])


# speedup vs baseline: 1.1182x; 1.0532x over previous
import functools

import jax
import jax.numpy as jnp
from jax import lax
from jax.experimental import pallas as pl
from jax.experimental.pallas import tpu as pltpu
from jax.experimental.pallas import tpu_sc as plsc

N = 10000
NP = 10240
E = 320000
D = 128
NQ = 8

NC = 2
NS = 16
NW = NC * NS
K = 120
NCH = 84
EPT = NCH * K
EP = NW * EPT
NIDX = 6
NROW = 3
UNROLL = 6
ROWS_PT = NP // NS

_MESH = dict(core_axis_name="c", subcore_axis_name="s",
             num_cores=NC, num_subcores=NS)


def _deg_body(dste_hbm, ones_hbm, zdeg_hbm, out_hbm, acc, dstb, ones_v):
    c = lax.axis_index("c")
    s = lax.axis_index("s")

    @pl.when(s == 0)
    def _():
        pltpu.sync_copy(zdeg_hbm, acc)

    wid = c * NS + s
    pltpu.sync_copy(dste_hbm.at[wid], dstb)
    pltpu.sync_copy(ones_hbm, ones_v)
    plsc.subcore_barrier()

    def body(j, carry):
        pltpu.sync_copy(ones_v, acc.at[dstb.at[j]], add=True)
        return carry

    lax.fori_loop(0, NCH, body, 0)
    plsc.subcore_barrier()

    @pl.when(s == 0)
    def _():
        pltpu.sync_copy(acc, out_hbm.at[pl.ds(c * NP, NP)])


def _spmm_body(y_hbm, srce_hbm, dste_hbm, z2d_hbm, out_hbm,
               acc, src_bufs, dst_bufs, isems, rows_bufs, gsems, ssems):
    c = lax.axis_index("c")
    s = lax.axis_index("s")
    wid = c * NS + s

    pltpu.sync_copy(z2d_hbm, acc.at[pl.ds(s * ROWS_PT, ROWS_PT)])
    plsc.subcore_barrier()

    ebase = wid * EPT
    idx_sets = tuple(zip(src_bufs, dst_bufs, isems))

    def fire_idx(ch, srcv, dstv, isem):
        pltpu.async_copy(srce_hbm.at[pl.ds(ebase + ch * K, K)], srcv, isem)
        pltpu.async_copy(dste_hbm.at[pl.ds(ebase + ch * K, K)], dstv, isem)

    def wait_idx(srcv, dstv, isem):
        pltpu.make_async_copy(srce_hbm.at[pl.ds(ebase, K)], srcv, isem).wait()
        pltpu.make_async_copy(dste_hbm.at[pl.ds(ebase, K)], dstv, isem).wait()

    def wait_scatter(b):
        pltpu.make_async_copy(rows_bufs[b], acc.at[idx_sets[0][1]],
                              ssems[b]).wait()

    fire_idx(0, *idx_sets[0])
    fire_idx(1, *idx_sets[1])
    wait_idx(*idx_sets[0])
    pltpu.async_copy(y_hbm.at[idx_sets[0][0]], rows_bufs[0], gsems[0])

    def body(i, carry):
        for j in range(UNROLL):
            ch = i * UNROLL + j
            b = j % NROW
            srcv, dstv, isem = idx_sets[j % NIDX]

            @pl.when(ch >= 2)
            def _():
                wait_scatter((j + 1) % NROW)

            @pl.when(ch + 1 < NCH)
            def _():
                wait_idx(*idx_sets[(j + 1) % NIDX])
                pltpu.async_copy(y_hbm.at[idx_sets[(j + 1) % NIDX][0]],
                                 rows_bufs[(j + 1) % NROW],
                                 gsems[(j + 1) % NROW])

            pltpu.make_async_copy(y_hbm.at[srcv], rows_bufs[b], gsems[b]).wait()
            pltpu.async_copy(rows_bufs[b], acc.at[dstv], ssems[b], add=True)

            @pl.when(ch + 2 < NCH)
            def _():
                fire_idx(ch + 2, *idx_sets[(j + 2) % NIDX])
        return carry

    lax.fori_loop(0, NCH // UNROLL, body, 0)
    wait_scatter((NCH - 2) % NROW)
    wait_scatter((NCH - 1) % NROW)
    plsc.subcore_barrier()

    pltpu.sync_copy(acc.at[pl.ds(s * ROWS_PT, ROWS_PT)],
                    out_hbm.at[c, pl.ds(s * ROWS_PT, ROWS_PT)])


@functools.cache
def _get_deg_kernel():
    return pl.kernel(
        _deg_body,
        out_type=jax.ShapeDtypeStruct((NC * NP,), jnp.float32),
        mesh=plsc.VectorSubcoreMesh(**_MESH),
        scratch_types=[
            pltpu.VMEM_SHARED((NP,), jnp.float32),
            pltpu.VMEM((NCH, K), jnp.int32),
            pltpu.VMEM((K,), jnp.float32),
        ],
    )


@functools.cache
def _get_spmm_kernel():
    return pl.kernel(
        _spmm_body,
        out_type=jax.ShapeDtypeStruct((NC, NP, D), jnp.float32),
        mesh=plsc.VectorSubcoreMesh(**_MESH),
        scratch_types=[
            pltpu.VMEM_SHARED((NP, D), jnp.float32),
            [pltpu.VMEM((K,), jnp.int32)] * NIDX,
            [pltpu.VMEM((K,), jnp.int32)] * NIDX,
            [pltpu.SemaphoreType.DMA] * NIDX,
            [pltpu.VMEM((K, D), jnp.float32)] * NROW,
            [pltpu.SemaphoreType.DMA] * NROW,
            [pltpu.SemaphoreType.DMA] * NROW,
        ],
    )


R = 1024


def _dinv(d0, d1):
    return lax.rsqrt(d0 + d1 + 1.0)


def _prep1_body(d0_ref, d1_ref, x_ref, w_ref, o_ref):
    dinv = _dinv(d0_ref[...], d1_ref[...])
    o_ref[...] = dinv * jnp.dot(x_ref[...], w_ref[...],
                                preferred_element_type=jnp.float32)


def _mid_body(d0_ref, d1_ref, a0_ref, a1_ref, y1_ref, b1_ref, w2_ref,
              q_ref, wm_ref, bm_ref, o_ref, ques_ref):
    dinv = _dinv(d0_ref[...], d1_ref[...])
    h1 = dinv * (a0_ref[...] + a1_ref[...] + y1_ref[...]) + b1_ref[...]
    h1 = jnp.maximum(h1, 0.0)
    o_ref[...] = dinv * jnp.dot(h1, w2_ref[...],
                                preferred_element_type=jnp.float32)

    @pl.when(pl.program_id(0) == 0)
    def _():
        ques_ref[...] = jnp.maximum(
            jnp.dot(q_ref[...], wm_ref[...],
                    preferred_element_type=jnp.float32) + bm_ref[...], 0.0)


def _final_body(d0_ref, d1_ref, a0_ref, a1_ref, y2_ref, b2_ref, o_ref):
    dinv = _dinv(d0_ref[...], d1_ref[...])
    o_ref[...] = dinv * (a0_ref[...] + a1_ref[...] + y2_ref[...]) + b2_ref[...]


def _col_spec():
    return pl.BlockSpec((R, 1), lambda i: (i, 0))


def _row_spec():
    return pl.BlockSpec((R, D), lambda i: (i, 0))


def _full_spec(shape):
    return pl.BlockSpec(shape, lambda i: (0,) * len(shape))


_prep1 = pl.pallas_call(
    _prep1_body,
    grid=(NP // R,),
    in_specs=[_col_spec(), _col_spec(), _row_spec(), _full_spec((D, D))],
    out_specs=_row_spec(),
    out_shape=jax.ShapeDtypeStruct((NP, D), jnp.float32),
)

_mid = pl.pallas_call(
    _mid_body,
    grid=(NP // R,),
    in_specs=[_col_spec(), _col_spec(), _row_spec(), _row_spec(), _row_spec(),
              _full_spec((1, D)), _full_spec((D, D)),
              _full_spec((NQ, D)), _full_spec((D, D)), _full_spec((1, D))],
    out_specs=[_row_spec(), _full_spec((NQ, D))],
    out_shape=[jax.ShapeDtypeStruct((NP, D), jnp.float32),
               jax.ShapeDtypeStruct((NQ, D), jnp.float32)],
)

_final = pl.pallas_call(
    _final_body,
    grid=(NP // R,),
    in_specs=[_col_spec(), _col_spec(), _row_spec(), _row_spec(), _row_spec(),
              _full_spec((1, D))],
    out_specs=_row_spec(),
    out_shape=jax.ShapeDtypeStruct((NP, D), jnp.float32),
)


def kernel(x, edge_index, W1, b1, W2, b2, q_emb, Wm, bm):
    pad = N + (jnp.arange(EP - E, dtype=jnp.int32) % (NP - N))
    srcf = jnp.concatenate([edge_index[0], pad])
    dstf = jnp.concatenate([edge_index[1], pad])
    dstp = dstf.reshape(NW, NCH, K)
    xp = jnp.pad(x, ((0, NP - N), (0, 0)))

    zdeg = jnp.zeros((NP,), jnp.float32)
    z2d = jnp.zeros((ROWS_PT, D), jnp.float32)
    ones_k = jnp.ones((K,), jnp.float32)

    degp = _get_deg_kernel()(dstp, ones_k, zdeg)
    d0 = degp[:NP].reshape(NP, 1)
    d1 = degp[NP:].reshape(NP, 1)

    y1 = _prep1(d0, d1, xp, W1)
    acc1 = _get_spmm_kernel()(y1, srcf, dstf, z2d)
    y2, ques = _mid(d0, d1, acc1[0], acc1[1], y1, b1.reshape(1, D), W2,
                    q_emb, Wm, bm.reshape(1, D))
    acc2 = _get_spmm_kernel()(y2, srcf, dstf, z2d)
    h2 = _final(d0, d1, acc2[0], acc2[1], y2, b2.reshape(1, D))
    return (ques, h2[:N])
